# baseline (speedup 1.0000x reference)
import functools

import jax
import jax.numpy as jnp
from jax import lax
from jax.experimental import pallas as pl
from jax.experimental.pallas import tpu as pltpu
from jax.experimental.pallas import tpu_sc as plsc

NUM_TABLE_ROWS = 30
EMBED_DIM = 64
NUM_INDICES = 4096 * 200
NUM_CORES = 2
NUM_SUBCORES = 16
NUM_WORKERS = NUM_CORES * NUM_SUBCORES
CHUNK = 400
NBUF = 2
NSPLIT = 2

_mesh = plsc.VectorSubcoreMesh(core_axis_name="c", subcore_axis_name="s")


@functools.lru_cache(maxsize=None)
def _make_embed_gather(nbatch):
    nrows = nbatch * 200
    r_per_w = nrows // NUM_WORKERS
    nchunk = r_per_w // CHUNK

    @functools.partial(
        pl.kernel,
        out_type=jax.ShapeDtypeStruct((nbatch, 200, EMBED_DIM), jnp.float32),
        mesh=_mesh,
        scratch_types=[
            pltpu.VMEM_SHARED((NUM_TABLE_ROWS, EMBED_DIM), jnp.float32),
            pltpu.VMEM((NUM_TABLE_ROWS * EMBED_DIM,), jnp.float32),
            [pltpu.VMEM((CHUNK,), jnp.int32) for _ in range(NBUF)],
            [pltpu.VMEM((CHUNK, EMBED_DIM), jnp.float32)
             for _ in range(NBUF)],
            [pltpu.SemaphoreType.DMA for _ in range(NBUF)],
            pltpu.SemaphoreType.DMA,
            [pltpu.SemaphoreType.DMA for _ in range(NBUF)],
        ],
    )
    def _embed_gather(idx_hbm, tb_hbm, out_hbm, tb_sh, tb1d_v, idx_v,
                      rows_v, i_s, g_s, s_s):
        sid = lax.axis_index("s")
        wid = sid * NUM_CORES + lax.axis_index("c")
        base = wid * r_per_w
        out2 = out_hbm.reshape(nrows, EMBED_DIM)

        @pl.when(sid == 0)
        def _():
            pltpu.sync_copy(tb_hbm, tb1d_v)
            for r in range(NUM_TABLE_ROWS):
                pltpu.sync_copy(tb1d_v.at[pl.ds(r * EMBED_DIM, EMBED_DIM)],
                                tb_sh.at[r])
        plsc.subcore_barrier()

        def chunk_off(g):
            return pl.multiple_of(base + g * CHUNK, 16)

        def start_idx(g, b):
            off = chunk_off(g)
            pltpu.async_copy(idx_hbm.at[pl.ds(off, CHUNK)], idx_v[b], i_s[b])

        for b in range(NBUF):
            start_idx(b, b)

        def process(g, b):
            off = chunk_off(g)
            pltpu.make_async_copy(idx_hbm.at[pl.ds(off, CHUNK)],
                                  idx_v[b], i_s[b]).wait()
            @pl.when(g + NBUF < nchunk)
            def _():
                start_idx(g + NBUF, b)
            @pl.when(g >= NBUF)
            def _():
                pltpu.make_async_copy(rows_v[b], out2.at[pl.ds(off, CHUNK)],
                                      s_s[b]).wait()
            pltpu.async_copy(tb_sh.at[idx_v[b]], rows_v[b], g_s).wait()
            pltpu.async_copy(rows_v[b], out2.at[pl.ds(off, CHUNK)], s_s[b])

        def loop_body(i, carry):
            for b in range(NBUF):
                process(i * NBUF + b, b)
            return carry

        lax.fori_loop(0, nchunk // NBUF, loop_body, 0)

        for b in range(NBUF):
            off = chunk_off(nchunk - NBUF + b)
            pltpu.make_async_copy(rows_v[b], out2.at[pl.ds(off, CHUNK)],
                                  s_s[b]).wait()

    return _embed_gather


def kernel(types, table):
    tb1d = table.reshape(-1)
    nb = types.shape[0] // NSPLIT
    fn = _make_embed_gather(nb)
    parts = [fn(types[i * nb:(i + 1) * nb].reshape(-1), tb1d)
             for i in range(NSPLIT)]
    return jnp.concatenate(parts, axis=0)


# speedup vs baseline: 1.3626x; 1.3626x over previous
import functools

import jax
import jax.numpy as jnp
from jax import lax
from jax.experimental import pallas as pl
from jax.experimental.pallas import tpu as pltpu
from jax.experimental.pallas import tpu_sc as plsc

NUM_TABLE_ROWS = 30
EMBED_DIM = 64
NUM_INDICES = 4096 * 200
NUM_CORES = 2
NUM_SUBCORES = 16
NUM_WORKERS = NUM_CORES * NUM_SUBCORES
CHUNK = 400
NBUF = 2

_mesh = plsc.VectorSubcoreMesh(core_axis_name="c", subcore_axis_name="s")


@functools.lru_cache(maxsize=None)
def _make_embed_gather(nbatch):
    nrows = nbatch * 200
    r_per_w = nrows // NUM_WORKERS
    nchunk = r_per_w // CHUNK

    @functools.partial(
        pl.kernel,
        out_type=jax.ShapeDtypeStruct((nbatch, 200, EMBED_DIM), jnp.float32),
        mesh=_mesh,
        scratch_types=[
            pltpu.VMEM_SHARED((NUM_TABLE_ROWS, EMBED_DIM), jnp.float32),
            pltpu.VMEM((NUM_TABLE_ROWS * EMBED_DIM,), jnp.float32),
            [pltpu.VMEM((CHUNK,), jnp.int32) for _ in range(NBUF)],
            [pltpu.VMEM((CHUNK, EMBED_DIM), jnp.float32)
             for _ in range(NBUF)],
            [pltpu.SemaphoreType.DMA for _ in range(NBUF)],
            pltpu.SemaphoreType.DMA,
            [pltpu.SemaphoreType.DMA for _ in range(NBUF)],
        ],
    )
    def _embed_gather(idx_hbm, tb_hbm, out_hbm, tb_sh, tb1d_v, idx_v,
                      rows_v, i_s, g_s, s_s):
        sid = lax.axis_index("s")
        wid = sid * NUM_CORES + lax.axis_index("c")
        base = wid * r_per_w
        out2 = out_hbm.reshape(nrows, EMBED_DIM)

        @pl.when(sid == 0)
        def _():
            pltpu.sync_copy(tb_hbm, tb1d_v)
            for r in range(NUM_TABLE_ROWS):
                pltpu.sync_copy(tb1d_v.at[pl.ds(r * EMBED_DIM, EMBED_DIM)],
                                tb_sh.at[r])
        plsc.subcore_barrier()

        def chunk_off(g):
            return pl.multiple_of(base + g * CHUNK, 16)

        def start_idx(g, b):
            off = chunk_off(g)
            pltpu.async_copy(idx_hbm.at[pl.ds(off, CHUNK)], idx_v[b], i_s[b])

        for b in range(NBUF):
            start_idx(b, b)

        def process(g, b):
            off = chunk_off(g)
            pltpu.make_async_copy(idx_hbm.at[pl.ds(off, CHUNK)],
                                  idx_v[b], i_s[b]).wait()
            @pl.when(g + NBUF < nchunk)
            def _():
                start_idx(g + NBUF, b)
            @pl.when(g >= NBUF)
            def _():
                pltpu.make_async_copy(rows_v[b], out2.at[pl.ds(off, CHUNK)],
                                      s_s[b]).wait()
            pltpu.async_copy(tb_sh.at[idx_v[b]], rows_v[b], g_s).wait()
            pltpu.async_copy(rows_v[b], out2.at[pl.ds(off, CHUNK)], s_s[b])

        def loop_body(i, carry):
            for b in range(NBUF):
                process(i * NBUF + b, b)
            return carry

        lax.fori_loop(0, nchunk // NBUF, loop_body, 0)

        for b in range(NBUF):
            off = chunk_off(nchunk - NBUF + b)
            pltpu.make_async_copy(rows_v[b], out2.at[pl.ds(off, CHUNK)],
                                  s_s[b]).wait()

    return _embed_gather


def kernel(types, table):
    fn = _make_embed_gather(types.shape[0])
    return fn(types.reshape(-1), table.reshape(-1))
